# baseline (speedup 1.0000x reference)
import jax
import jax.numpy as jnp
from jax import lax
from jax.experimental import pallas as pl
from jax.experimental.pallas import tpu as pltpu
from jax.experimental.pallas import tpu_sc as plsc

D = 128
K = 8
L = 16
NC, NS = 2, 16
NW = NC * NS
CHUNK = 16
NBUF = 4
CH_W = 98
N_CHUNKS_TOTAL = NW * CH_W
M_PAD = N_CHUNKS_TOTAL * CHUNK
IDX_PER_CHUNK = CHUNK * K


def _pool_body(x_hbm, idx_hbm, out_hbm, idx_v, rows_v, out_v, *gsems):
    wid = lax.axis_index("s") * NC + lax.axis_index("c")
    base_chunk = wid * CH_W
    base_row = base_chunk * CHUNK
    m_out = out_hbm.shape[0]
    n_real_chunks = m_out // CHUNK

    pltpu.sync_copy(
        idx_hbm.at[pl.ds(base_chunk * IDX_PER_CHUNK, CH_W * IDX_PER_CHUNK)],
        idx_v,
    )

    def start_gather(c, buf):
        pltpu.async_copy(
            x_hbm.at[idx_v.at[pl.ds(c * IDX_PER_CHUNK, IDX_PER_CHUNK)]],
            rows_v.at[buf],
            gsems[buf],
        )

    def wait_gather(c, buf):
        pltpu.make_async_copy(
            x_hbm.at[idx_v.at[pl.ds(c * IDX_PER_CHUNK, IDX_PER_CHUNK)]],
            rows_v.at[buf],
            gsems[buf],
        ).wait()

    def compute_and_store(c, buf):
        rows = rows_v.at[buf]

        def row_body(r, carry):
            b = r * K
            for dcol in range(D // L):
                sl = pl.ds(dcol * L, L)
                acc = rows[b, sl]
                for k in range(1, K):
                    acc = jnp.maximum(acc, rows[b + k, sl])
                out_v[r, sl] = acc
            return carry

        lax.fori_loop(0, CHUNK, row_body, 0)

        @pl.when(base_chunk + c < n_real_chunks)
        def _():
            pltpu.sync_copy(
                out_v, out_hbm.at[pl.ds(base_row + c * CHUNK, CHUNK)]
            )

    for b in range(NBUF - 1):
        start_gather(b, b)

    n_groups = CH_W // NBUF

    def group_body(g, carry):
        c0 = NBUF * g
        for b in range(NBUF):
            c = c0 + b
            wait_gather(c, b)

            @pl.when(c + NBUF - 1 < CH_W)
            def _():
                start_gather(c + NBUF - 1, (b + NBUF - 1) % NBUF)

            compute_and_store(c, b)
        return carry

    lax.fori_loop(0, n_groups, group_body, 0)

    for t in range(n_groups * NBUF, CH_W):
        wait_gather(t, t % NBUF)
        compute_and_store(t, t % NBUF)


def kernel(x, indices, weights):
    del weights
    m = indices.shape[0]
    v = x.shape[0]
    flat = jnp.ravel(indices.astype(jnp.int32))
    n_pad = M_PAD * K - flat.shape[0]
    pad_idx = jnp.arange(n_pad, dtype=jnp.int32) % v
    idx_flat = jnp.concatenate([flat, pad_idx])

    mesh = plsc.VectorSubcoreMesh(core_axis_name="c", subcore_axis_name="s")
    out = pl.kernel(
        _pool_body,
        out_type=jax.ShapeDtypeStruct((m, D), jnp.float32),
        mesh=mesh,
        scratch_types=[
            pltpu.VMEM((CH_W * IDX_PER_CHUNK,), jnp.int32),
            pltpu.VMEM((NBUF, IDX_PER_CHUNK, D), jnp.float32),
            pltpu.VMEM((CHUNK, D), jnp.float32),
        ] + [pltpu.SemaphoreType.DMA] * NBUF,
        compiler_params=pltpu.CompilerParams(use_tc_tiling_on_sc=False),
    )(x, idx_flat)
    return out


# speedup vs baseline: 1.0832x; 1.0832x over previous
import jax
import jax.numpy as jnp
from jax import lax
from jax.experimental import pallas as pl
from jax.experimental.pallas import tpu as pltpu
from jax.experimental.pallas import tpu_sc as plsc

D = 128
K = 8
L = 16
NC, NS = 2, 16
NW = NC * NS
CHUNK = 16
NBUF = 4
NOBUF = 2
CH_W = 98
N_CHUNKS_TOTAL = NW * CH_W
M_PAD = N_CHUNKS_TOTAL * CHUNK
IDX_PER_CHUNK = CHUNK * K


def _pool_body(x_hbm, idx_hbm, out_hbm, idx_v, rows_v, out_v, *sems):
    gsems = sems[:NBUF]
    osems = sems[NBUF:]
    wid = lax.axis_index("s") * NC + lax.axis_index("c")
    base_chunk = wid * CH_W
    base_row = base_chunk * CHUNK
    m_out = out_hbm.shape[0]
    n_real_chunks = m_out // CHUNK

    pltpu.sync_copy(
        idx_hbm.at[pl.ds(base_chunk * IDX_PER_CHUNK, CH_W * IDX_PER_CHUNK)],
        idx_v,
    )

    def start_gather(c, buf):
        pltpu.async_copy(
            x_hbm.at[idx_v.at[pl.ds(c * IDX_PER_CHUNK, IDX_PER_CHUNK)]],
            rows_v.at[buf],
            gsems[buf],
        )

    def wait_gather(c, buf):
        pltpu.make_async_copy(
            x_hbm.at[idx_v.at[pl.ds(c * IDX_PER_CHUNK, IDX_PER_CHUNK)]],
            rows_v.at[buf],
            gsems[buf],
        ).wait()

    def compute_and_store(c, buf, ob):
        rows = rows_v.at[buf]
        outb = out_v.at[ob]

        @pl.when((c >= NOBUF) & (base_chunk + c - NOBUF < n_real_chunks))
        def _():
            pltpu.make_async_copy(
                outb,
                out_hbm.at[pl.ds(base_row + (c - NOBUF) * CHUNK, CHUNK)],
                osems[ob],
            ).wait()

        def row_body(r, carry):
            b = r * K
            for dcol in range(D // L):
                sl = pl.ds(dcol * L, L)
                acc = rows[b, sl]
                for k in range(1, K):
                    acc = jnp.maximum(acc, rows[b + k, sl])
                outb[r, sl] = acc
            return carry

        lax.fori_loop(0, CHUNK, row_body, 0)

        @pl.when(base_chunk + c < n_real_chunks)
        def _():
            pltpu.async_copy(
                outb, out_hbm.at[pl.ds(base_row + c * CHUNK, CHUNK)],
                osems[ob],
            )

    for b in range(NBUF - 1):
        start_gather(b, b)

    n_groups = CH_W // NBUF

    def group_body(g, carry):
        c0 = NBUF * g
        for b in range(NBUF):
            c = c0 + b
            wait_gather(c, b)

            @pl.when(c + NBUF - 1 < CH_W)
            def _():
                start_gather(c + NBUF - 1, (b + NBUF - 1) % NBUF)

            compute_and_store(c, b, b % NOBUF)
        return carry

    lax.fori_loop(0, n_groups, group_body, 0)

    for t in range(n_groups * NBUF, CH_W):
        wait_gather(t, t % NBUF)
        compute_and_store(t, t % NBUF, t % NOBUF)

    for t in range(CH_W - NOBUF, CH_W):
        @pl.when(base_chunk + t < n_real_chunks)
        def _():
            pltpu.make_async_copy(
                out_v.at[t % NOBUF],
                out_hbm.at[pl.ds(base_row + t * CHUNK, CHUNK)],
                osems[t % NOBUF],
            ).wait()


def kernel(x, indices, weights):
    del weights
    m = indices.shape[0]
    v = x.shape[0]
    flat = jnp.ravel(indices.astype(jnp.int32))
    n_pad = M_PAD * K - flat.shape[0]
    pad_idx = jnp.arange(n_pad, dtype=jnp.int32) % v
    idx_flat = jnp.concatenate([flat, pad_idx])

    mesh = plsc.VectorSubcoreMesh(core_axis_name="c", subcore_axis_name="s")
    out = pl.kernel(
        _pool_body,
        out_type=jax.ShapeDtypeStruct((m, D), jnp.float32),
        mesh=mesh,
        scratch_types=[
            pltpu.VMEM((CH_W * IDX_PER_CHUNK,), jnp.int32),
            pltpu.VMEM((NBUF, IDX_PER_CHUNK, D), jnp.float32),
            pltpu.VMEM((NOBUF, CHUNK, D), jnp.float32),
        ] + [pltpu.SemaphoreType.DMA] * (NBUF + NOBUF),
    )(x, idx_flat)
    return out
